# Initial kernel scaffold; baseline (speedup 1.0000x reference)
import functools

import jax
import jax.numpy as jnp
from jax import lax
from jax.experimental import pallas as pl
from jax.experimental.pallas import tpu as pltpu
from jax.experimental.pallas import tpu_sc as plsc

N = 10000
E = 320000
D = 128
L = 4
M = 10
MAX_N = 100

NUM_TILES = 32
EDGES_PER_TILE = E // NUM_TILES
CHUNK = 125
NCHUNKS = EDGES_PER_TILE // CHUNK
ROWS_PER_TILE = N // 16

_HIGH = lax.Precision.HIGHEST


_sc_mesh = plsc.VectorSubcoreMesh(core_axis_name="c", subcore_axis_name="s")


@functools.partial(
    pl.kernel,
    mesh=_sc_mesh,
    out_type=jax.ShapeDtypeStruct((2, 16, ROWS_PER_TILE, D), jnp.float32),
    scratch_types=[
        pltpu.VMEM((NCHUNKS, CHUNK), jnp.int32),
        pltpu.VMEM((NCHUNKS, CHUNK), jnp.int32),
        pltpu.VMEM((CHUNK, D), jnp.float32),
        pltpu.VMEM_SHARED((N, D), jnp.float32),
        pltpu.SemaphoreType.DMA,
    ],
)
def _sc_agg(h_hbm, src_hbm, dst_hbm, zero_hbm, out_hbm,
            src_v, dst_v, rows_v, agg_sh, sem):
    cid = lax.axis_index("c")
    sid = lax.axis_index("s")
    wid = cid * 16 + sid
    pltpu.sync_copy(src_hbm.at[wid], src_v)
    pltpu.sync_copy(dst_hbm.at[wid], dst_v)
    pltpu.sync_copy(zero_hbm, agg_sh.at[pl.ds(sid * ROWS_PER_TILE, ROWS_PER_TILE)])
    plsc.subcore_barrier()

    def chunk(j, carry):
        pltpu.async_copy(h_hbm.at[src_v.at[j]], rows_v, sem).wait()
        pltpu.sync_copy(rows_v, agg_sh.at[dst_v.at[j]], add=True)
        return carry

    lax.fori_loop(0, NCHUNKS, chunk, 0)
    plsc.subcore_barrier()
    pltpu.sync_copy(agg_sh.at[pl.ds(sid * ROWS_PER_TILE, ROWS_PER_TILE)],
                    out_hbm.at[cid, sid])


def _bn(z, g, b):
    mu = jnp.mean(z, axis=0, keepdims=True)
    var = jnp.mean((z - mu) ** 2, axis=0, keepdims=True)
    return (z - mu) * lax.rsqrt(var + 1e-5) * g + b


def _embed_body(x_ref, w_ref, b_ref, g_ref, bb_ref, out_ref):
    h = jnp.dot(x_ref[...], w_ref[...], precision=_HIGH,
                preferred_element_type=jnp.float32) + b_ref[...]
    out_ref[...] = _bn(h, g_ref[...], bb_ref[...])


def _gin_body(h_ref, a0_ref, a1_ref, w1_ref, b1_ref, w2_ref, b2_ref,
              g_ref, bb_ref, out_ref):
    z = h_ref[...] + a0_ref[...] + a1_ref[...]
    t = jnp.maximum(jnp.dot(z, w1_ref[...], precision=_HIGH,
                            preferred_element_type=jnp.float32) + b1_ref[...], 0.0)
    t = jnp.dot(t, w2_ref[...], precision=_HIGH,
                preferred_element_type=jnp.float32) + b2_ref[...]
    out_ref[...] = _bn(t, g_ref[...], bb_ref[...])


def _out_body(h_ref, w0_ref, b0_ref, w1_ref, b1_ref, w2t_ref, b2_ref,
              mu_ref, std_ref):
    h = h_ref[...]
    ys = []
    for m in range(M):
        y = jnp.maximum(jnp.dot(h, w0_ref[m], precision=_HIGH,
                                preferred_element_type=jnp.float32) + b0_ref[m], 0.0)
        y = jnp.maximum(jnp.dot(y, w1_ref[m], precision=_HIGH,
                                preferred_element_type=jnp.float32) + b1_ref[m], 0.0)
        yv = jnp.sum(y * w2t_ref[m], axis=1, keepdims=True) + b2_ref[m]
        ys.append(yv)
    acc = ys[0]
    for yv in ys[1:]:
        acc = acc + yv
    mu = acc * (1.0 / M)
    acc2 = (ys[0] - mu) ** 2
    for yv in ys[1:]:
        acc2 = acc2 + (yv - mu) ** 2
    mu_ref[...] = mu
    std_ref[...] = jnp.sqrt(acc2 * (1.0 / M))


def _dense(body, out_shapes, *args):
    return pl.pallas_call(
        body,
        out_shape=out_shapes,
    )(*args)


def kernel(x, edge_index, input_idx, lin_embed_W, lin_embed_b, input_bn_g,
           input_bn_b, gin_W1, gin_b1, gin_W2, gin_b2, gin_bn_g, gin_bn_b,
           out_W0, out_b0, out_W1, out_b1, out_W2, out_b2):
    del input_idx

    src = edge_index[0].reshape(NUM_TILES, NCHUNKS, CHUNK)
    dst = edge_index[1].reshape(NUM_TILES, NCHUNKS, CHUNK)
    zeros = jnp.zeros((ROWS_PER_TILE, D), jnp.float32)

    r = lambda v: v.reshape(1, -1)

    h = _dense(_embed_body, jax.ShapeDtypeStruct((N, D), jnp.float32),
               x, lin_embed_W, r(lin_embed_b), r(input_bn_g), r(input_bn_b))

    for i in range(L):
        parts = _sc_agg(h, src, dst, zeros)
        a0 = parts[0].reshape(N, D)
        a1 = parts[1].reshape(N, D)
        h = _dense(_gin_body, jax.ShapeDtypeStruct((N, D), jnp.float32),
                   h, a0, a1, gin_W1[i], r(gin_b1[i]), gin_W2[i], r(gin_b2[i]),
                   r(gin_bn_g[i]), r(gin_bn_b[i]))

    w2t = out_W2.transpose(0, 2, 1)
    b2 = out_b2.reshape(M, 1, 1)
    b0 = out_b0.reshape(M, 1, D)
    b1 = out_b1.reshape(M, 1, D)
    mu, std = _dense(
        _out_body,
        (jax.ShapeDtypeStruct((N, 1), jnp.float32),
         jax.ShapeDtypeStruct((N, 1), jnp.float32)),
        h, out_W0, b0, out_W1, b1, w2t, b2)
    return (mu.reshape(-1, MAX_N, 1), std.reshape(-1, MAX_N, 1))


# speedup vs baseline: 6.7959x; 6.7959x over previous
import functools

import jax
import jax.numpy as jnp
from jax import lax
from jax.experimental import pallas as pl
from jax.experimental.pallas import tpu as pltpu
from jax.experimental.pallas import tpu_sc as plsc

N = 10000
E = 320000
D = 128
L = 4
M = 10
MAX_N = 100

NUM_TILES = 32
EDGES_PER_TILE = E // NUM_TILES
CHUNK = 125
NCHUNKS = EDGES_PER_TILE // CHUNK
ROWS_PER_TILE = N // 16


@functools.lru_cache(maxsize=1)
def _build_sc_agg():
    mesh = plsc.VectorSubcoreMesh(core_axis_name="c", subcore_axis_name="s")

    @functools.partial(
        pl.kernel,
        mesh=mesh,
        out_type=jax.ShapeDtypeStruct((2, 16, ROWS_PER_TILE, D), jnp.float32),
        scratch_types=[
            pltpu.VMEM((NCHUNKS, CHUNK), jnp.int32),
            pltpu.VMEM((NCHUNKS, CHUNK), jnp.int32),
            pltpu.VMEM((CHUNK, D), jnp.float32),
            pltpu.VMEM_SHARED((N, D), jnp.float32),
            pltpu.SemaphoreType.DMA,
        ],
    )
    def sc_agg(h_hbm, src_hbm, dst_hbm, zero_hbm, out_hbm,
               src_v, dst_v, rows_v, agg_sh, sem):
        cid = lax.axis_index("c")
        sid = lax.axis_index("s")
        wid = cid * 16 + sid
        pltpu.sync_copy(src_hbm.at[wid], src_v)
        pltpu.sync_copy(dst_hbm.at[wid], dst_v)
        pltpu.sync_copy(zero_hbm, agg_sh.at[pl.ds(sid * ROWS_PER_TILE, ROWS_PER_TILE)])
        plsc.subcore_barrier()

        def chunk(j, carry):
            pltpu.async_copy(h_hbm.at[src_v.at[j]], rows_v, sem).wait()
            pltpu.sync_copy(rows_v, agg_sh.at[dst_v.at[j]], add=True)
            return carry

        lax.fori_loop(0, NCHUNKS, chunk, 0)
        plsc.subcore_barrier()
        pltpu.sync_copy(agg_sh.at[pl.ds(sid * ROWS_PER_TILE, ROWS_PER_TILE)],
                        out_hbm.at[cid, sid])

    return sc_agg


def _sc_agg(h, src, dst, zeros):
    return _build_sc_agg()(h, src, dst, zeros)


def _bn(z, g, b):
    mu = jnp.mean(z, axis=0, keepdims=True)
    var = jnp.mean((z - mu) ** 2, axis=0, keepdims=True)
    return (z - mu) / jnp.sqrt(var + 1e-5) * g + b


def _embed_body(x_ref, w_ref, b_ref, g_ref, bb_ref, out_ref):
    h = jnp.dot(x_ref[...], w_ref[...],
                preferred_element_type=jnp.float32) + b_ref[...]
    out_ref[...] = _bn(h, g_ref[...], bb_ref[...])


def _gin_body(h_ref, a0_ref, a1_ref, w1_ref, b1_ref, w2_ref, b2_ref,
              g_ref, bb_ref, out_ref):
    z = h_ref[...] + a0_ref[...] + a1_ref[...]
    t = jnp.maximum(jnp.dot(z, w1_ref[...],
                            preferred_element_type=jnp.float32) + b1_ref[...], 0.0)
    t = jnp.dot(t, w2_ref[...],
                preferred_element_type=jnp.float32) + b2_ref[...]
    out_ref[...] = _bn(t, g_ref[...], bb_ref[...])


def _out_body(h_ref, w0_ref, b0_ref, w1_ref, b1_ref, w2t_ref, b2_ref,
              mu_ref, std_ref):
    h = h_ref[...]
    acc = jnp.zeros((h.shape[0], 1), jnp.float32)
    acc2 = jnp.zeros((h.shape[0], 1), jnp.float32)
    for m in range(M):
        y = jnp.maximum(jnp.dot(h, w0_ref[m],
                                preferred_element_type=jnp.float32) + b0_ref[m], 0.0)
        y = jnp.maximum(jnp.dot(y, w1_ref[m],
                                preferred_element_type=jnp.float32) + b1_ref[m], 0.0)
        yv = jnp.sum(y * w2t_ref[m], axis=1, keepdims=True) + b2_ref[m]
        acc = acc + yv
        acc2 = acc2 + yv * yv
    mu = acc * (1.0 / M)
    var = jnp.maximum(acc2 * (1.0 / M) - mu * mu, 0.0)
    mu_ref[...] = mu
    std_ref[...] = jnp.sqrt(var)


def _dense(body, out_shapes, *args):
    return pl.pallas_call(
        body,
        out_shape=out_shapes,
    )(*args)


def kernel(x, edge_index, input_idx, lin_embed_W, lin_embed_b, input_bn_g,
           input_bn_b, gin_W1, gin_b1, gin_W2, gin_b2, gin_bn_g, gin_bn_b,
           out_W0, out_b0, out_W1, out_b1, out_W2, out_b2):
    del input_idx

    src = edge_index[0].reshape(NUM_TILES, NCHUNKS, CHUNK)
    dst = edge_index[1].reshape(NUM_TILES, NCHUNKS, CHUNK)
    zeros = jnp.zeros((ROWS_PER_TILE, D), jnp.float32)

    r = lambda v: v.reshape(1, -1)

    h = _dense(_embed_body, jax.ShapeDtypeStruct((N, D), jnp.float32),
               x, lin_embed_W, r(lin_embed_b), r(input_bn_g), r(input_bn_b))

    for i in range(L):
        parts = _sc_agg(h, src, dst, zeros)
        a0 = parts[0].reshape(N, D)
        a1 = parts[1].reshape(N, D)
        h = _dense(_gin_body, jax.ShapeDtypeStruct((N, D), jnp.float32),
                   h, a0, a1, gin_W1[i], r(gin_b1[i]), gin_W2[i], r(gin_b2[i]),
                   r(gin_bn_g[i]), r(gin_bn_b[i]))

    w2t = out_W2.transpose(0, 2, 1)
    b2 = out_b2.reshape(M, 1, 1)
    b0 = out_b0.reshape(M, 1, D)
    b1 = out_b1.reshape(M, 1, D)
    BLK = 2000
    full = lambda s: pl.BlockSpec(s, lambda i: (0,) * len(s))
    mu, std = pl.pallas_call(
        _out_body,
        grid=(N // BLK,),
        in_specs=[
            pl.BlockSpec((BLK, D), lambda i: (i, 0)),
            full((M, D, D)), full((M, 1, D)),
            full((M, D, D)), full((M, 1, D)),
            full((M, 1, D)), full((M, 1, 1)),
        ],
        out_specs=(pl.BlockSpec((BLK, 1), lambda i: (i, 0)),
                   pl.BlockSpec((BLK, 1), lambda i: (i, 0))),
        out_shape=(jax.ShapeDtypeStruct((N, 1), jnp.float32),
                   jax.ShapeDtypeStruct((N, 1), jnp.float32)),
    )(h, out_W0, b0, out_W1, b1, w2t, b2)
    return (mu.reshape(-1, MAX_N, 1), std.reshape(-1, MAX_N, 1))


# speedup vs baseline: 6.8008x; 1.0007x over previous
import functools

import jax
import jax.numpy as jnp
from jax import lax
from jax.experimental import pallas as pl
from jax.experimental.pallas import tpu as pltpu
from jax.experimental.pallas import tpu_sc as plsc

N = 10000
E = 320000
D = 128
L = 4
M = 10
MAX_N = 100

NUM_TILES = 32
EDGES_PER_TILE = E // NUM_TILES
CHUNK = 125
NCHUNKS = EDGES_PER_TILE // CHUNK
ROWS_PER_TILE = N // 16


@functools.lru_cache(maxsize=1)
def _build_sc_agg():
    mesh = plsc.VectorSubcoreMesh(core_axis_name="c", subcore_axis_name="s")

    @functools.partial(
        pl.kernel,
        mesh=mesh,
        out_type=jax.ShapeDtypeStruct((2, 16, ROWS_PER_TILE, D), jnp.float32),
        scratch_types=[
            pltpu.VMEM((NCHUNKS, CHUNK), jnp.int32),
            pltpu.VMEM((NCHUNKS, CHUNK), jnp.int32),
            pltpu.VMEM((CHUNK, D), jnp.float32),
            pltpu.VMEM_SHARED((N, D), jnp.float32),
            pltpu.SemaphoreType.DMA,
        ],
    )
    def sc_agg(h_hbm, src_hbm, dst_hbm, zero_hbm, out_hbm,
               src_v, dst_v, r0, agg_sh, s0):
        cid = lax.axis_index("c")
        sid = lax.axis_index("s")
        wid = cid * 16 + sid
        pltpu.sync_copy(src_hbm.at[wid], src_v)
        pltpu.sync_copy(dst_hbm.at[wid], dst_v)
        pltpu.sync_copy(zero_hbm, agg_sh.at[pl.ds(sid * ROWS_PER_TILE, ROWS_PER_TILE)])
        plsc.subcore_barrier()

        def chunk(j, carry):
            pltpu.async_copy(h_hbm.at[src_v.at[j]], r0, s0).wait()
            pltpu.sync_copy(r0, agg_sh.at[dst_v.at[j]], add=True)
            return carry

        lax.fori_loop(0, NCHUNKS, chunk, 0)
        plsc.subcore_barrier()
        pltpu.sync_copy(agg_sh.at[pl.ds(sid * ROWS_PER_TILE, ROWS_PER_TILE)],
                        out_hbm.at[cid, sid])

    return sc_agg


def _sc_agg(h, src, dst, zeros):
    return _build_sc_agg()(h, src, dst, zeros)


def _bn(z, g, b):
    mu = jnp.mean(z, axis=0, keepdims=True)
    var = jnp.mean((z - mu) ** 2, axis=0, keepdims=True)
    return (z - mu) / jnp.sqrt(var + 1e-5) * g + b


def _embed_body(x_ref, w_ref, b_ref, g_ref, bb_ref, out_ref):
    h = jnp.dot(x_ref[...], w_ref[...],
                preferred_element_type=jnp.float32) + b_ref[...]
    out_ref[...] = _bn(h, g_ref[...], bb_ref[...])


def _gin_body(h_ref, a0_ref, a1_ref, w1_ref, b1_ref, w2_ref, b2_ref,
              g_ref, bb_ref, out_ref):
    z = h_ref[...] + a0_ref[...] + a1_ref[...]
    t = jnp.maximum(jnp.dot(z, w1_ref[...],
                            preferred_element_type=jnp.float32) + b1_ref[...], 0.0)
    t = jnp.dot(t, w2_ref[...],
                preferred_element_type=jnp.float32) + b2_ref[...]
    out_ref[...] = _bn(t, g_ref[...], bb_ref[...])


def _out_body(h_ref, w0_ref, b0_ref, w1_ref, b1_ref, w2t_ref, b2_ref,
              mu_ref, std_ref):
    h = h_ref[...]
    acc = jnp.zeros((h.shape[0], 1), jnp.float32)
    acc2 = jnp.zeros((h.shape[0], 1), jnp.float32)
    for m in range(M):
        y = jnp.maximum(jnp.dot(h, w0_ref[m],
                                preferred_element_type=jnp.float32) + b0_ref[m], 0.0)
        y = jnp.maximum(jnp.dot(y, w1_ref[m],
                                preferred_element_type=jnp.float32) + b1_ref[m], 0.0)
        yv = jnp.sum(y * w2t_ref[m], axis=1, keepdims=True) + b2_ref[m]
        acc = acc + yv
        acc2 = acc2 + yv * yv
    mu = acc * (1.0 / M)
    var = jnp.maximum(acc2 * (1.0 / M) - mu * mu, 0.0)
    mu_ref[...] = mu
    std_ref[...] = jnp.sqrt(var)


def _dense(body, out_shapes, *args):
    return pl.pallas_call(
        body,
        out_shape=out_shapes,
    )(*args)


def kernel(x, edge_index, input_idx, lin_embed_W, lin_embed_b, input_bn_g,
           input_bn_b, gin_W1, gin_b1, gin_W2, gin_b2, gin_bn_g, gin_bn_b,
           out_W0, out_b0, out_W1, out_b1, out_W2, out_b2):
    del input_idx

    src = edge_index[0].reshape(NUM_TILES, NCHUNKS, CHUNK)
    dst = edge_index[1].reshape(NUM_TILES, NCHUNKS, CHUNK)
    zeros = jnp.zeros((ROWS_PER_TILE, D), jnp.float32)

    r = lambda v: v.reshape(1, -1)

    h = _dense(_embed_body, jax.ShapeDtypeStruct((N, D), jnp.float32),
               x, lin_embed_W, r(lin_embed_b), r(input_bn_g), r(input_bn_b))

    for i in range(L):
        parts = _sc_agg(h, src, dst, zeros)
        a0 = parts[0].reshape(N, D)
        a1 = parts[1].reshape(N, D)
        h = _dense(_gin_body, jax.ShapeDtypeStruct((N, D), jnp.float32),
                   h, a0, a1, gin_W1[i], r(gin_b1[i]), gin_W2[i], r(gin_b2[i]),
                   r(gin_bn_g[i]), r(gin_bn_b[i]))

    w2t = out_W2.transpose(0, 2, 1)
    b2 = out_b2.reshape(M, 1, 1)
    b0 = out_b0.reshape(M, 1, D)
    b1 = out_b1.reshape(M, 1, D)
    BLK = 2000
    full = lambda s: pl.BlockSpec(s, lambda i: (0,) * len(s))
    mu, std = pl.pallas_call(
        _out_body,
        grid=(N // BLK,),
        in_specs=[
            pl.BlockSpec((BLK, D), lambda i: (i, 0)),
            full((M, D, D)), full((M, 1, D)),
            full((M, D, D)), full((M, 1, D)),
            full((M, 1, D)), full((M, 1, 1)),
        ],
        out_specs=(pl.BlockSpec((BLK, 1), lambda i: (i, 0)),
                   pl.BlockSpec((BLK, 1), lambda i: (i, 0))),
        out_shape=(jax.ShapeDtypeStruct((N, 1), jnp.float32),
                   jax.ShapeDtypeStruct((N, 1), jnp.float32)),
    )(h, out_W0, b0, out_W1, b1, w2t, b2)
    return (mu.reshape(-1, MAX_N, 1), std.reshape(-1, MAX_N, 1))
